# baseline (speedup 1.0000x reference)
import functools

import jax
import jax.numpy as jnp
from jax import lax
from jax.experimental import pallas as pl
from jax.experimental.pallas import tpu as pltpu
from jax.experimental.pallas import tpu_sc as plsc

B = 16384
K = 32
NC = 2
NS = 16
NW = NC * NS
BPW = B // NW
CHUNK = 128
NCHUNK = BPW // CHUNK
SUPW = 128

ROWS = 1000000
NSTRIPE = (ROWS + 127) // 128
PACKED_ROWS = NSTRIPE * 32
SPW = (NSTRIPE + NW - 1) // NW


def _relayout_body(src_hbm, dst_hbm, in_v, out_v, sem_in, sem_out):
    c = lax.axis_index("c")
    s = lax.axis_index("s")
    wid = s * NC + c

    lane = lax.iota(jnp.int32, 16)
    klo = lane
    khi = lane + 16

    def stripe(t, carry):
        sg = wid * SPW + t

        @pl.when(sg < NSTRIPE)
        def _():
            r0 = sg * 128
            pltpu.async_copy(
                src_hbm.at[:, pl.ds(pl.multiple_of(r0, 128), 128)],
                in_v, sem_in).wait()
            for r in range(128):
                rr = jnp.full((16,), r, jnp.int32)
                a = plsc.load_gather(in_v, [klo, rr])
                b = plsc.load_gather(in_v, [khi, rr])
                col = (r % 4) * 32
                out_v[r // 4, pl.ds(col, 16)] = a
                out_v[r // 4, pl.ds(col + 16, 16)] = b
            pltpu.async_copy(
                out_v, dst_hbm.at[pl.ds(pl.multiple_of(sg * 32, 32), 32)],
                sem_out).wait()

        return carry

    lax.fori_loop(0, SPW, stripe, 0)


def _gather_body(usup_hbm, uoff_hbm, isup_hbm, ioff_hbm, w4_hbm, h4_hbm,
                 out_hbm, usup_v, uoff_v, isup_v, ioff_v, ubuf, vbuf, out_v,
                 usem, vsem):
    c = lax.axis_index("c")
    s = lax.axis_index("s")
    wid = s * NC + c
    base = wid * BPW

    pltpu.sync_copy(usup_hbm.at[wid], usup_v)
    pltpu.sync_copy(uoff_hbm.at[wid], uoff_v)
    pltpu.sync_copy(isup_hbm.at[wid], isup_v)
    pltpu.sync_copy(ioff_hbm.at[wid], ioff_v)

    lane = lax.iota(jnp.int32, 16)

    def fire(j):
        cu = pltpu.async_copy(w4_hbm.at[usup_v.at[j]], ubuf.at[j % 2], usem)
        cv = pltpu.async_copy(h4_hbm.at[isup_v.at[j]], vbuf.at[j % 2], vsem)
        return cu, cv

    inflight = fire(0)

    for j in range(NCHUNK):
        cu, cv = inflight
        cu.wait()
        cv.wait()
        if j + 1 < NCHUNK:
            inflight = fire(j + 1)
        jb = j % 2

        def group(g, carry):
            i0 = g * 16
            uoff = uoff_v[j, pl.ds(i0, 16)]
            ioff = ioff_v[j, pl.ds(i0, 16)]
            acc = jnp.zeros((16,), jnp.float32)
            for di in range(16):
                i = i0 + di
                pu = uoff[di]
                pv = ioff[di]
                u0 = ubuf[jb, i, pl.ds(pu, 16)]
                u1 = ubuf[jb, i, pl.ds(pu + 16, 16)]
                v0 = vbuf[jb, i, pl.ds(pv, 16)]
                v1 = vbuf[jb, i, pl.ds(pv + 16, 16)]
                p = u0 * v0 + u1 * v1
                total = jnp.sum(p, axis=0)
                acc = jnp.where(lane == di, total, acc)
            out_v[pl.ds(pl.multiple_of(j * CHUNK + i0, 16), 16)] = acc
            return carry

        lax.fori_loop(0, CHUNK // 16, group, 0)

    pltpu.sync_copy(out_v, out_hbm.at[pl.ds(base, BPW)])


@functools.partial(jax.jit, donate_argnums=())
def kernel(x, W, H):
    xi = x.astype(jnp.int32)
    uidx = xi[:, 0]
    iidx = xi[:, 1]
    usup = jnp.right_shift(uidx, 2).reshape(NW, NCHUNK, CHUNK)
    uoff = (jnp.bitwise_and(uidx, 3) * K).reshape(NW, NCHUNK, CHUNK)
    isup = jnp.right_shift(iidx, 2).reshape(NW, NCHUNK, CHUNK)
    ioff = (jnp.bitwise_and(iidx, 3) * K).reshape(NW, NCHUNK, CHUNK)

    mesh = plsc.VectorSubcoreMesh(core_axis_name="c", subcore_axis_name="s")

    relayout = functools.partial(
        pl.kernel,
        mesh=mesh,
        compiler_params=pltpu.CompilerParams(needs_layout_passes=False),
        out_type=jax.ShapeDtypeStruct((PACKED_ROWS, SUPW), jnp.float32),
        scratch_types=[
            pltpu.VMEM((K, 128), jnp.float32),
            pltpu.VMEM((32, SUPW), jnp.float32),
            pltpu.SemaphoreType.DMA,
            pltpu.SemaphoreType.DMA,
        ],
    )(_relayout_body)
    W4 = relayout(W.T)
    H4 = relayout(H.T)

    run = functools.partial(
        pl.kernel,
        mesh=mesh,
        compiler_params=pltpu.CompilerParams(needs_layout_passes=False),
        out_type=jax.ShapeDtypeStruct((B,), jnp.float32),
        scratch_types=[
            pltpu.VMEM((NCHUNK, CHUNK), jnp.int32),
            pltpu.VMEM((NCHUNK, CHUNK), jnp.int32),
            pltpu.VMEM((NCHUNK, CHUNK), jnp.int32),
            pltpu.VMEM((NCHUNK, CHUNK), jnp.int32),
            pltpu.VMEM((2, CHUNK, SUPW), jnp.float32),
            pltpu.VMEM((2, CHUNK, SUPW), jnp.float32),
            pltpu.VMEM((BPW,), jnp.float32),
            pltpu.SemaphoreType.DMA,
            pltpu.SemaphoreType.DMA,
        ],
    )(_gather_body)
    return run(usup, uoff, isup, ioff, W4, H4)


# speedup vs baseline: 1.3445x; 1.3445x over previous
import functools

import jax
import jax.numpy as jnp
from jax import lax
from jax.experimental import pallas as pl
from jax.experimental.pallas import tpu as pltpu
from jax.experimental.pallas import tpu_sc as plsc

B = 16384
K = 32
NC = 2
NS = 16
NW = NC * NS
BPW = B // NW
CHUNK = 128
NCHUNK = BPW // CHUNK
SUPW = 128

ROWS = 1000000
NSTRIPE = (ROWS + 127) // 128
PACKED_ROWS = NSTRIPE * 32
SPW = 246


def _relayout_body(wt_hbm, ht_hbm, w4_hbm, h4_hbm,
                   win0, win1, hin0, hin1, wout0, wout1, hout0, hout1,
                   wis0, wis1, his0, his1, wos0, wos1, hos0, hos1):
    c = lax.axis_index("c")
    s = lax.axis_index("s")
    wid = s * NC + c

    lane = lax.iota(jnp.int32, 16)
    klo = lane
    khi = lane + 16

    def live(t):
        return jnp.logical_and(t < SPW, wid * SPW + t < NSTRIPE)

    def fire_in(t, src, buf, sem):
        @pl.when(live(t))
        def _():
            r0 = (wid * SPW + t) * 128
            pltpu.async_copy(
                src.at[:, pl.ds(pl.multiple_of(r0, 128), 128)], buf, sem)

    def wait_in(t, src, buf, sem):
        @pl.when(live(t))
        def _():
            pltpu.make_async_copy(
                src.at[:, pl.ds(0, 128)], buf, sem).wait()

    def transpose(buf, obuf):
        for r in range(128):
            rr = jnp.full((16,), r, jnp.int32)
            a = plsc.load_gather(buf, [klo, rr])
            b = plsc.load_gather(buf, [khi, rr])
            col = (r % 4) * 32
            obuf[r // 4, pl.ds(col, 16)] = a
            obuf[r // 4, pl.ds(col + 16, 16)] = b

    def fire_out(t, dst, obuf, sem):
        @pl.when(live(t))
        def _():
            p0 = (wid * SPW + t) * 32
            pltpu.async_copy(
                obuf, dst.at[pl.ds(pl.multiple_of(p0, 32), 32)], sem)

    def wait_out(t, dst, obuf, sem):
        @pl.when(jnp.logical_and(t >= 0, live(t)))
        def _():
            pltpu.make_async_copy(
                obuf, dst.at[pl.ds(0, 32)], sem).wait()

    def do_slot(t, src, dst, buf, obuf, isem, osem):
        wait_in(t, src, buf, isem)

        @pl.when(live(t))
        def _():
            transpose(buf, obuf)

        fire_out(t, dst, obuf, osem)

    fire_in(0, wt_hbm, win0, wis0)
    fire_in(0, ht_hbm, hin0, his0)

    def pair(tt, carry):
        a = tt * 2
        b = a + 1
        fire_in(b, wt_hbm, win1, wis1)
        fire_in(b, ht_hbm, hin1, his1)
        wait_out(a - 2, w4_hbm, wout0, wos0)
        wait_out(a - 2, h4_hbm, hout0, hos0)
        do_slot(a, wt_hbm, w4_hbm, win0, wout0, wis0, wos0)
        do_slot(a, ht_hbm, h4_hbm, hin0, hout0, his0, hos0)
        fire_in(a + 2, wt_hbm, win0, wis0)
        fire_in(a + 2, ht_hbm, hin0, his0)
        wait_out(b - 2, w4_hbm, wout1, wos1)
        wait_out(b - 2, h4_hbm, hout1, hos1)
        do_slot(b, wt_hbm, w4_hbm, win1, wout1, wis1, wos1)
        do_slot(b, ht_hbm, h4_hbm, hin1, hout1, his1, hos1)
        return carry

    lax.fori_loop(0, SPW // 2, pair, 0)
    wait_out(SPW - 2, w4_hbm, wout0, wos0)
    wait_out(SPW - 2, h4_hbm, hout0, hos0)
    wait_out(SPW - 1, w4_hbm, wout1, wos1)
    wait_out(SPW - 1, h4_hbm, hout1, hos1)


def _gather_body(usup_hbm, uoff_hbm, isup_hbm, ioff_hbm, w4_hbm, h4_hbm,
                 out_hbm, usup_v, uoff_v, isup_v, ioff_v, ubuf, vbuf, out_v,
                 usem, vsem):
    c = lax.axis_index("c")
    s = lax.axis_index("s")
    wid = s * NC + c
    base = wid * BPW

    pltpu.sync_copy(usup_hbm.at[wid], usup_v)
    pltpu.sync_copy(uoff_hbm.at[wid], uoff_v)
    pltpu.sync_copy(isup_hbm.at[wid], isup_v)
    pltpu.sync_copy(ioff_hbm.at[wid], ioff_v)

    lane = lax.iota(jnp.int32, 16)

    def fire(j):
        cu = pltpu.async_copy(w4_hbm.at[usup_v.at[j]], ubuf.at[j % 2], usem)
        cv = pltpu.async_copy(h4_hbm.at[isup_v.at[j]], vbuf.at[j % 2], vsem)
        return cu, cv

    inflight = fire(0)

    for j in range(NCHUNK):
        cu, cv = inflight
        cu.wait()
        cv.wait()
        if j + 1 < NCHUNK:
            inflight = fire(j + 1)
        jb = j % 2

        def group(g, carry):
            i0 = g * 16
            uoff = uoff_v[j, pl.ds(i0, 16)]
            ioff = ioff_v[j, pl.ds(i0, 16)]
            acc = jnp.zeros((16,), jnp.float32)
            for di in range(16):
                i = i0 + di
                pu = uoff[di]
                pv = ioff[di]
                u0 = ubuf[jb, i, pl.ds(pu, 16)]
                u1 = ubuf[jb, i, pl.ds(pu + 16, 16)]
                v0 = vbuf[jb, i, pl.ds(pv, 16)]
                v1 = vbuf[jb, i, pl.ds(pv + 16, 16)]
                p = u0 * v0 + u1 * v1
                total = jnp.sum(p, axis=0)
                acc = jnp.where(lane == di, total, acc)
            out_v[pl.ds(pl.multiple_of(j * CHUNK + i0, 16), 16)] = acc
            return carry

        lax.fori_loop(0, CHUNK // 16, group, 0)

    pltpu.sync_copy(out_v, out_hbm.at[pl.ds(base, BPW)])


@functools.partial(jax.jit, donate_argnums=())
def kernel(x, W, H):
    xi = x.astype(jnp.int32)
    uidx = xi[:, 0]
    iidx = xi[:, 1]
    usup = jnp.right_shift(uidx, 2).reshape(NW, NCHUNK, CHUNK)
    uoff = (jnp.bitwise_and(uidx, 3) * K).reshape(NW, NCHUNK, CHUNK)
    isup = jnp.right_shift(iidx, 2).reshape(NW, NCHUNK, CHUNK)
    ioff = (jnp.bitwise_and(iidx, 3) * K).reshape(NW, NCHUNK, CHUNK)

    mesh = plsc.VectorSubcoreMesh(core_axis_name="c", subcore_axis_name="s")

    relayout = functools.partial(
        pl.kernel,
        mesh=mesh,
        compiler_params=pltpu.CompilerParams(needs_layout_passes=False),
        out_type=[jax.ShapeDtypeStruct((PACKED_ROWS, SUPW), jnp.float32),
                  jax.ShapeDtypeStruct((PACKED_ROWS, SUPW), jnp.float32)],
        scratch_types=(
            [pltpu.VMEM((K, 128), jnp.float32)] * 4
            + [pltpu.VMEM((32, SUPW), jnp.float32)] * 4
            + [pltpu.SemaphoreType.DMA] * 8
        ),
    )(_relayout_body)
    W4, H4 = relayout(W.T, H.T)

    run = functools.partial(
        pl.kernel,
        mesh=mesh,
        compiler_params=pltpu.CompilerParams(needs_layout_passes=False),
        out_type=jax.ShapeDtypeStruct((B,), jnp.float32),
        scratch_types=[
            pltpu.VMEM((NCHUNK, CHUNK), jnp.int32),
            pltpu.VMEM((NCHUNK, CHUNK), jnp.int32),
            pltpu.VMEM((NCHUNK, CHUNK), jnp.int32),
            pltpu.VMEM((NCHUNK, CHUNK), jnp.int32),
            pltpu.VMEM((2, CHUNK, SUPW), jnp.float32),
            pltpu.VMEM((2, CHUNK, SUPW), jnp.float32),
            pltpu.VMEM((BPW,), jnp.float32),
            pltpu.SemaphoreType.DMA,
            pltpu.SemaphoreType.DMA,
        ],
    )(_gather_body)
    return run(usup, uoff, isup, ioff, W4, H4)


# speedup vs baseline: 3.1487x; 2.3419x over previous
import functools

import jax
import jax.numpy as jnp
from jax import lax
from jax.experimental import pallas as pl
from jax.experimental.pallas import tpu as pltpu
from jax.experimental.pallas import tpu_sc as plsc

B = 16384
K = 32
NC = 2
NS = 16
NW = NC * NS
BPW = B // NW
G = 16
NG = BPW // G


def _body(uidx_hbm, iidx_hbm, w_hbm, h_hbm, out_hbm,
          uidx_v, iidx_v, u0buf, u1buf, v0buf, v1buf, out_v,
          usem0, usem1, vsem0, vsem1):
    c = lax.axis_index("c")
    s = lax.axis_index("s")
    wid = s * NC + c
    base = wid * BPW

    pltpu.sync_copy(uidx_hbm.at[wid], uidx_v)
    pltpu.sync_copy(iidx_hbm.at[wid], iidx_v)

    lane = lax.iota(jnp.int32, 16)

    def fire(g, ubuf, vbuf, usem, vsem):
        i0 = g * G
        uvec = uidx_v[pl.ds(pl.multiple_of(i0, G), G)]
        ivec = iidx_v[pl.ds(pl.multiple_of(i0, G), G)]
        for di in range(G):
            pltpu.async_copy(w_hbm.at[uvec[di]], ubuf.at[di], usem)
            pltpu.async_copy(h_hbm.at[ivec[di]], vbuf.at[di], vsem)

    def drain_and_compute(g, ubuf, vbuf, usem, vsem):
        for di in range(G):
            pltpu.make_async_copy(w_hbm.at[0], ubuf.at[di], usem).wait()
            pltpu.make_async_copy(h_hbm.at[0], vbuf.at[di], vsem).wait()
        acc = jnp.zeros((16,), jnp.float32)
        for di in range(G):
            u0 = ubuf[di, pl.ds(0, 16)]
            u1 = ubuf[di, pl.ds(16, 16)]
            v0 = vbuf[di, pl.ds(0, 16)]
            v1 = vbuf[di, pl.ds(16, 16)]
            p = u0 * v0 + u1 * v1
            total = jnp.sum(p, axis=0)
            acc = jnp.where(lane == di, total, acc)
        out_v[pl.ds(pl.multiple_of(g * G, G), G)] = acc

    def pair(t, carry):
        ga = t * 2
        fire(ga, u0buf, v0buf, usem0, vsem0)

        @pl.when(t > 0)
        def _():
            drain_and_compute(ga - 1, u1buf, v1buf, usem1, vsem1)

        fire(ga + 1, u1buf, v1buf, usem1, vsem1)
        drain_and_compute(ga, u0buf, v0buf, usem0, vsem0)
        return carry

    lax.fori_loop(0, NG // 2, pair, 0)
    drain_and_compute(NG - 1, u1buf, v1buf, usem1, vsem1)

    pltpu.sync_copy(out_v, out_hbm.at[pl.ds(base, BPW)])


@functools.partial(jax.jit, donate_argnums=())
def kernel(x, W, H):
    xi = x.astype(jnp.int32)
    uidx = xi[:, 0].reshape(NW, BPW)
    iidx = xi[:, 1].reshape(NW, BPW)

    mesh = plsc.VectorSubcoreMesh(core_axis_name="c", subcore_axis_name="s")
    run = functools.partial(
        pl.kernel,
        mesh=mesh,
        compiler_params=pltpu.CompilerParams(needs_layout_passes=False),
        out_type=jax.ShapeDtypeStruct((B,), jnp.float32),
        scratch_types=[
            pltpu.VMEM((BPW,), jnp.int32),
            pltpu.VMEM((BPW,), jnp.int32),
            pltpu.VMEM((G, K), jnp.float32),
            pltpu.VMEM((G, K), jnp.float32),
            pltpu.VMEM((G, K), jnp.float32),
            pltpu.VMEM((G, K), jnp.float32),
            pltpu.VMEM((BPW,), jnp.float32),
            pltpu.SemaphoreType.DMA,
            pltpu.SemaphoreType.DMA,
            pltpu.SemaphoreType.DMA,
            pltpu.SemaphoreType.DMA,
        ],
    )(_body)
    return run(uidx, iidx, W, H)
